# Initial kernel scaffold; baseline (speedup 1.0000x reference)
import jax
import jax.numpy as jnp
from jax.experimental import pallas as pl


def kernel(flat, cu_seqlens, params):
    raise NotImplementedError("write your pallas kernel here")


# speedup vs baseline: 2.6374x; 2.6374x over previous
import jax
import jax.numpy as jnp
from jax import lax
from jax.experimental import pallas as pl
from jax.experimental.pallas import tpu as pltpu

T = 4096
B = 16
WIN = 512
MAXN = 513
TS = 512
NT = T // TS
PADR = T + 2 * TS

_f32 = jnp.float32
_i32 = jnp.int32


def _dot(a, b):
    return lax.dot_general(a, b, (((1,), (0,)), ((), ())),
                           preferred_element_type=_f32)


def _dotT(a, b):
    return lax.dot_general(a, b, (((0,), (0,)), ((), ())),
                           preferred_element_type=_f32)


def _dotR(a, b):
    return lax.dot_general(a, b, (((0,), (1,)), ((), ())),
                           preferred_element_type=_f32)


def _prep_kernel(cu_ref, flat_ref, rank_W_ref, rank_b_ref,
                 kds_W1_ref, kds_b1_ref, kds_W2_ref, kds_b2_ref,
                 magc_ref, magr_ref, segc_ref, segr_ref,
                 S_ref, ohl_ref, tab_ref):
    flat = flat_ref[...]
    magc_ref[...] = _dot(flat, rank_W_ref[...]) + rank_b_ref[...]
    magr_ref[...] = jnp.zeros((1, PADR), _f32)
    magr_ref[0:1, TS:TS + T] = _dotR(rank_W_ref[...], flat) + rank_b_ref[...]

    iota_t = lax.broadcasted_iota(_i32, (1, T), 1)
    iota_n = lax.broadcasted_iota(_i32, (1, MAXN), 1)
    for s in range(B):
        c = cu_ref[s]
        n = cu_ref[s + 1]
        S_ref[s:s + 1, :] = ((iota_t >= c) & (iota_t < n)).astype(_f32)
        ohl_ref[s:s + 1, :] = (iota_n == (n - c)).astype(_f32)

    ar_col = lax.broadcasted_iota(_i32, (B, 1), 0).astype(_f32)
    segc_ref[...] = _dotT(S_ref[...], ar_col)
    segr_ref[...] = jnp.full((1, PADR), -1.0, _f32)
    segr_ref[0:1, TS:TS + T] = _dotT(ar_col, S_ref[...])

    tab_ref[...] = _dot(jax.nn.relu(kds_W1_ref[...] + kds_b1_ref[...]),
                        kds_W2_ref[...]) + kds_b2_ref[...]


def _rank_kernel(magc_ref, segc_ref, magr_ref, segr_ref, rank_ref):
    i = pl.program_id(0)
    mag_c = magc_ref[...]
    seg_c = segc_ref[...]
    iidx = i * TS + lax.broadcasted_iota(_i32, (TS, TS), 0)
    cnt = jnp.zeros((TS, 1), _f32)
    for k in range(3):
        off = (i + k) * TS
        mag_r = magr_ref[0:1, pl.ds(off, TS)]
        seg_r = segr_ref[0:1, pl.ds(off, TS)]
        jidx = (i + k - 1) * TS + lax.broadcasted_iota(_i32, (TS, TS), 1)
        less = mag_r < mag_c
        tie = (mag_r == mag_c) & (jidx < iidx)
        m = ((less | tie) & (seg_r == seg_c)).astype(_f32)
        cnt = cnt + jnp.sum(m, axis=1, keepdims=True)
    rank_ref[...] = cnt.astype(_i32)


def _ds_kernel(flat_ref, rank_ref, S_ref, tab_ref,
               vds_W1_ref, vds_b1_ref, vds_W2_ref, vds_b2_ref,
               y2ds_ref):
    i = pl.program_id(0)

    @pl.when(i == 0)
    def _():
        y2ds_ref[...] = jnp.zeros((B, 128), _f32)

    fl = flat_ref[...]
    P = (rank_ref[...] == lax.broadcasted_iota(_i32, (TS, WIN), 1)).astype(_f32)
    kds_tok = _dot(P, tab_ref[...])
    vds_tok = _dot(jax.nn.relu(_dot(fl, vds_W1_ref[...]) + vds_b1_ref[...]),
                   vds_W2_ref[...]) + vds_b2_ref[...]
    y2ds_ref[...] += _dot(S_ref[...], vds_tok * kds_tok)


def _enc_ds_kernel(y2ds_ref, eds_W1_ref, eds_b1_ref, eds_W2_ref, eds_b2_ref,
                   km_W1b_ref, zds_ref, cseg_ref):
    z_ds = _dot(jax.nn.relu(_dot(y2ds_ref[...], eds_W1_ref[...]) +
                            eds_b1_ref[...]),
                eds_W2_ref[...]) + eds_b2_ref[...]
    zds_ref[...] = z_ds
    cseg_ref[...] = _dot(z_ds, km_W1b_ref[...])


def _main_kernel(flat_ref, rank_ref, S_ref, zds_ref, cseg_ref, ohl_ref,
                 km_W1a_ref, km_b1_ref, km_W2_ref, km_b2_ref,
                 vm_W1a_ref, vm_W1b_ref, vm_b1_ref, vm_W2_ref, vm_b2_ref,
                 em_W1a_ref, em_W1b_ref, em_b1_ref, em_W2_ref, em_b2_ref,
                 out_ref, y2_scr):
    i = pl.program_id(0)

    @pl.when(i == 0)
    def _():
        y2_scr[...] = jnp.zeros((B, 64), _f32)

    fl = flat_ref[...]
    St = S_ref[...]
    P = (rank_ref[...] == lax.broadcasted_iota(_i32, (TS, WIN), 1)).astype(_f32)
    g = jax.nn.relu(_dot(P, km_W1a_ref[...]) + _dotT(St, cseg_ref[...])
                    + km_b1_ref[...])
    y_key = _dot(g, km_W2_ref[...]) + km_b2_ref[...]
    z_tok = _dotT(St, zds_ref[...])
    h = jax.nn.relu(_dot(fl, vm_W1a_ref[...]) + _dot(z_tok, vm_W1b_ref[...])
                    + vm_b1_ref[...])
    y_val = _dot(h, vm_W2_ref[...]) + vm_b2_ref[...]
    y2_scr[...] += _dot(St, y_val * y_key)

    @pl.when(i == NT - 1)
    def _():
        len_part = _dot(ohl_ref[...], em_W1b_ref[...])
        hE = jax.nn.relu(_dot(y2_scr[...], em_W1a_ref[...]) + len_part
                         + em_b1_ref[...])
        out_ref[...] = _dot(hE, em_W2_ref[...]) + em_b2_ref[...]


def _vm(block=None, imap=None):
    if block is None:
        return pl.BlockSpec(memory_space=pltpu.VMEM)
    return pl.BlockSpec(block, imap, memory_space=pltpu.VMEM)


def kernel(flat, cu_seqlens, params):
    p = params
    r2 = lambda b: b.reshape(1, -1)
    cu = cu_seqlens.astype(_i32)

    magc, magr, segc, segr, S, ohl, kds_tab = pl.pallas_call(
        _prep_kernel,
        out_shape=(jax.ShapeDtypeStruct((T, 1), _f32),
                   jax.ShapeDtypeStruct((1, PADR), _f32),
                   jax.ShapeDtypeStruct((T, 1), _f32),
                   jax.ShapeDtypeStruct((1, PADR), _f32),
                   jax.ShapeDtypeStruct((B, T), _f32),
                   jax.ShapeDtypeStruct((B, MAXN), _f32),
                   jax.ShapeDtypeStruct((WIN, 128), _f32)),
        in_specs=[pl.BlockSpec(memory_space=pltpu.SMEM)] + [_vm()] * 7,
        out_specs=(_vm(),) * 7,
    )(cu, flat, p["rank_W"], r2(p["rank_b"]),
      p["key_ds"]["W1"][:WIN], r2(p["key_ds"]["b1"]),
      p["key_ds"]["W2"], r2(p["key_ds"]["b2"]))

    rank = pl.pallas_call(
        _rank_kernel,
        grid=(NT,),
        out_shape=jax.ShapeDtypeStruct((T, 1), _i32),
        in_specs=[_vm((TS, 1), lambda i: (i, 0)),
                  _vm((TS, 1), lambda i: (i, 0)),
                  _vm(), _vm()],
        out_specs=_vm((TS, 1), lambda i: (i, 0)),
    )(magc, segc, magr, segr)

    y2ds = pl.pallas_call(
        _ds_kernel,
        grid=(NT,),
        out_shape=jax.ShapeDtypeStruct((B, 128), _f32),
        in_specs=[_vm((TS, 128), lambda i: (i, 0)),
                  _vm((TS, 1), lambda i: (i, 0)),
                  _vm((B, TS), lambda i: (0, i)),
                  _vm(), _vm(), _vm(), _vm(), _vm()],
        out_specs=_vm((B, 128), lambda i: (0, 0)),
    )(flat, rank, S, kds_tab,
      p["val_ds"]["W1"], r2(p["val_ds"]["b1"]),
      p["val_ds"]["W2"], r2(p["val_ds"]["b2"]))

    z_ds, c_seg = pl.pallas_call(
        _enc_ds_kernel,
        out_shape=(jax.ShapeDtypeStruct((B, 128), _f32),
                   jax.ShapeDtypeStruct((B, 352), _f32)),
        in_specs=[_vm()] * 6,
        out_specs=(_vm(), _vm()),
    )(y2ds, p["enc_ds"]["W1"], r2(p["enc_ds"]["b1"]),
      p["enc_ds"]["W2"], r2(p["enc_ds"]["b2"]), p["key_main"]["W1"][MAXN:])

    return pl.pallas_call(
        _main_kernel,
        grid=(NT,),
        out_shape=jax.ShapeDtypeStruct((B, 64), _f32),
        in_specs=[_vm((TS, 128), lambda i: (i, 0)),
                  _vm((TS, 1), lambda i: (i, 0)),
                  _vm((B, TS), lambda i: (0, i)),
                  _vm(), _vm(), _vm()] + [_vm()] * 14,
        out_specs=_vm((B, 64), lambda i: (0, 0)),
        scratch_shapes=[pltpu.VMEM((B, 64), _f32)],
    )(flat, rank, S, z_ds, c_seg, ohl,
      p["key_main"]["W1"][:WIN], r2(p["key_main"]["b1"]),
      p["key_main"]["W2"], r2(p["key_main"]["b2"]),
      p["val_main"]["W1"][:128], p["val_main"]["W1"][128:],
      r2(p["val_main"]["b1"]), p["val_main"]["W2"], r2(p["val_main"]["b2"]),
      p["enc_main"]["W1"][:64], p["enc_main"]["W1"][64:],
      r2(p["enc_main"]["b1"]), p["enc_main"]["W2"], r2(p["enc_main"]["b2"]))


# speedup vs baseline: 2.8170x; 1.0681x over previous
import jax
import jax.numpy as jnp
from jax import lax
from jax.experimental import pallas as pl
from jax.experimental.pallas import tpu as pltpu

T = 4096
B = 16
WIN = 512
MAXN = 513
TS = 512
NT = T // TS
PADR = T + 2 * TS

_f32 = jnp.float32
_bf16 = jnp.bfloat16
_i32 = jnp.int32


def _dot(a, b):
    return lax.dot_general(a, b, (((1,), (0,)), ((), ())),
                           preferred_element_type=_f32)


def _dotT(a, b):
    return lax.dot_general(a, b, (((0,), (0,)), ((), ())),
                           preferred_element_type=_f32)


def _dotR(a, b):
    return lax.dot_general(a, b, (((0,), (1,)), ((), ())),
                           preferred_element_type=_f32)


def _prep_kernel(cu_ref, flat_ref, rank_W_ref, rank_b_ref,
                 kds_W1_ref, kds_b1_ref, kds_W2_ref, kds_b2_ref,
                 magc_ref, magr_ref, segc_ref, segr_ref,
                 S_ref, ohl_ref, tab_ref):
    flat = flat_ref[...]
    mag = _dot(flat, rank_W_ref[...]) + rank_b_ref[...]
    magc_ref[...] = mag
    magr_ref[...] = jnp.zeros((1, PADR), _f32)
    magr_ref[0:1, TS:TS + T] = mag.reshape(1, T)

    iota_t = lax.broadcasted_iota(_i32, (1, T), 1)
    iota_n = lax.broadcasted_iota(_i32, (1, MAXN), 1)
    for s in range(B):
        c = cu_ref[s]
        n = cu_ref[s + 1]
        S_ref[s:s + 1, :] = ((iota_t >= c) & (iota_t < n)).astype(_f32)
        ohl_ref[s:s + 1, :] = (iota_n == (n - c)).astype(_f32)

    ar_col = lax.broadcasted_iota(_i32, (B, 1), 0).astype(_f32)
    segc_ref[...] = _dotT(S_ref[...], ar_col)
    segr_ref[...] = jnp.full((1, PADR), -1.0, _f32)
    segr_ref[0:1, TS:TS + T] = _dotT(ar_col, S_ref[...])

    tab_ref[...] = (_dot(jax.nn.relu(kds_W1_ref[...] + kds_b1_ref[...]),
                         kds_W2_ref[...]) + kds_b2_ref[...]).astype(_bf16)


def _rank_kernel(magc_ref, segc_ref, magr_ref, segr_ref, rank_ref):
    i = pl.program_id(0)
    mag_c = magc_ref[...]
    seg_c = segc_ref[...]
    tri = (lax.broadcasted_iota(_i32, (TS, TS), 1) <
           lax.broadcasted_iota(_i32, (TS, TS), 0))
    ones = jnp.ones((TS, 1), _bf16)
    cnt = jnp.zeros((TS, 1), _f32)
    for k in range(3):
        off = (i + k) * TS
        mag_r = magr_ref[0:1, pl.ds(off, TS)]
        seg_r = segr_ref[0:1, pl.ds(off, TS)]
        less = mag_r < mag_c
        if k == 0:
            cm = less | (mag_r == mag_c)
        elif k == 1:
            cm = less | ((mag_r == mag_c) & tri)
        else:
            cm = less
        m = (cm & (seg_r == seg_c)).astype(_bf16)
        cnt = cnt + _dot(m, ones)
    rank_ref[...] = cnt.astype(_i32)


def _ds_kernel(flat_ref, rank_ref, S_ref, tab_ref,
               vds_W1_ref, vds_b1_ref, vds_W2_ref, vds_b2_ref,
               eds_W1_ref, eds_b1_ref, eds_W2_ref, eds_b2_ref,
               km_W1b_ref, zds_ref, cseg_ref, y2ds_scr):
    i = pl.program_id(0)

    @pl.when(i == 0)
    def _():
        y2ds_scr[...] = jnp.zeros((B, 128), _f32)

    fl = flat_ref[...]
    P = (rank_ref[...] ==
         lax.broadcasted_iota(_i32, (TS, WIN), 1)).astype(_bf16)
    kds_tok = _dot(P, tab_ref[...])
    vds_tok = _dot(jax.nn.relu(_dot(fl, vds_W1_ref[...]) + vds_b1_ref[...]),
                   vds_W2_ref[...]) + vds_b2_ref[...]
    y2ds_scr[...] += _dot(S_ref[...], vds_tok * kds_tok)

    @pl.when(i == NT - 1)
    def _():
        z_ds = _dot(jax.nn.relu(_dot(y2ds_scr[...], eds_W1_ref[...]) +
                                eds_b1_ref[...]),
                    eds_W2_ref[...]) + eds_b2_ref[...]
        zds_ref[...] = z_ds
        cseg_ref[...] = _dot(z_ds, km_W1b_ref[...])


def _main_kernel(flat_ref, rank_ref, S_ref, zds_ref, cseg_ref, ohl_ref,
                 km_W1a_ref, km_b1_ref, km_W2_ref, km_b2_ref,
                 vm_W1a_ref, vm_W1b_ref, vm_b1_ref, vm_W2_ref, vm_b2_ref,
                 em_W1a_ref, em_W1b_ref, em_b1_ref, em_W2_ref, em_b2_ref,
                 out_ref, y2_scr):
    i = pl.program_id(0)

    @pl.when(i == 0)
    def _():
        y2_scr[...] = jnp.zeros((B, 64), _f32)

    fl = flat_ref[...]
    St = S_ref[...]
    P = (rank_ref[...] ==
         lax.broadcasted_iota(_i32, (TS, WIN), 1)).astype(_bf16)
    g = jax.nn.relu(_dot(P, km_W1a_ref[...]) + _dotT(St, cseg_ref[...])
                    + km_b1_ref[...])
    y_key = _dot(g, km_W2_ref[...]) + km_b2_ref[...]
    z_tok = _dotT(St, zds_ref[...])
    h = jax.nn.relu(_dot(fl, vm_W1a_ref[...]) + _dot(z_tok, vm_W1b_ref[...])
                    + vm_b1_ref[...])
    y_val = _dot(h, vm_W2_ref[...]) + vm_b2_ref[...]
    y2_scr[...] += _dot(St, y_val * y_key)

    @pl.when(i == NT - 1)
    def _():
        len_part = _dot(ohl_ref[...], em_W1b_ref[...])
        hE = jax.nn.relu(_dot(y2_scr[...], em_W1a_ref[...]) + len_part
                         + em_b1_ref[...])
        out_ref[...] = _dot(hE, em_W2_ref[...]) + em_b2_ref[...]


def _vm(block=None, imap=None):
    if block is None:
        return pl.BlockSpec(memory_space=pltpu.VMEM)
    return pl.BlockSpec(block, imap, memory_space=pltpu.VMEM)


def kernel(flat, cu_seqlens, params):
    p = params
    r2 = lambda b: b.reshape(1, -1)
    cu = cu_seqlens.astype(_i32)

    magc, magr, segc, segr, S, ohl, kds_tab = pl.pallas_call(
        _prep_kernel,
        out_shape=(jax.ShapeDtypeStruct((T, 1), _f32),
                   jax.ShapeDtypeStruct((1, PADR), _f32),
                   jax.ShapeDtypeStruct((T, 1), _f32),
                   jax.ShapeDtypeStruct((1, PADR), _f32),
                   jax.ShapeDtypeStruct((B, T), _f32),
                   jax.ShapeDtypeStruct((B, MAXN), _f32),
                   jax.ShapeDtypeStruct((WIN, 128), _bf16)),
        in_specs=[pl.BlockSpec(memory_space=pltpu.SMEM)] + [_vm()] * 7,
        out_specs=(_vm(),) * 7,
    )(cu, flat, p["rank_W"], r2(p["rank_b"]),
      p["key_ds"]["W1"][:WIN], r2(p["key_ds"]["b1"]),
      p["key_ds"]["W2"], r2(p["key_ds"]["b2"]))

    rank = pl.pallas_call(
        _rank_kernel,
        grid=(NT,),
        out_shape=jax.ShapeDtypeStruct((T, 1), _i32),
        in_specs=[_vm((TS, 1), lambda i: (i, 0)),
                  _vm((TS, 1), lambda i: (i, 0)),
                  _vm(), _vm()],
        out_specs=_vm((TS, 1), lambda i: (i, 0)),
    )(magc, segc, magr, segr)

    z_ds, c_seg = pl.pallas_call(
        _ds_kernel,
        grid=(NT,),
        out_shape=(jax.ShapeDtypeStruct((B, 128), _f32),
                   jax.ShapeDtypeStruct((B, 352), _f32)),
        in_specs=[_vm((TS, 128), lambda i: (i, 0)),
                  _vm((TS, 1), lambda i: (i, 0)),
                  _vm((B, TS), lambda i: (0, i)),
                  _vm()] + [_vm()] * 9,
        out_specs=(_vm((B, 128), lambda i: (0, 0)),
                   _vm((B, 352), lambda i: (0, 0))),
        scratch_shapes=[pltpu.VMEM((B, 128), _f32)],
    )(flat, rank, S, kds_tab,
      p["val_ds"]["W1"], r2(p["val_ds"]["b1"]),
      p["val_ds"]["W2"], r2(p["val_ds"]["b2"]),
      p["enc_ds"]["W1"], r2(p["enc_ds"]["b1"]),
      p["enc_ds"]["W2"], r2(p["enc_ds"]["b2"]),
      p["key_main"]["W1"][MAXN:])

    return pl.pallas_call(
        _main_kernel,
        grid=(NT,),
        out_shape=jax.ShapeDtypeStruct((B, 64), _f32),
        in_specs=[_vm((TS, 128), lambda i: (i, 0)),
                  _vm((TS, 1), lambda i: (i, 0)),
                  _vm((B, TS), lambda i: (0, i)),
                  _vm(), _vm(), _vm()] + [_vm()] * 14,
        out_specs=_vm((B, 64), lambda i: (0, 0)),
        scratch_shapes=[pltpu.VMEM((B, 64), _f32)],
    )(flat, rank, S, z_ds, c_seg, ohl,
      p["key_main"]["W1"][:WIN].astype(_bf16), r2(p["key_main"]["b1"]),
      p["key_main"]["W2"], r2(p["key_main"]["b2"]),
      p["val_main"]["W1"][:128], p["val_main"]["W1"][128:],
      r2(p["val_main"]["b1"]), p["val_main"]["W2"], r2(p["val_main"]["b2"]),
      p["enc_main"]["W1"][:64], p["enc_main"]["W1"][64:],
      r2(p["enc_main"]["b1"]), p["enc_main"]["W2"], r2(p["enc_main"]["b2"]))


# speedup vs baseline: 2.8717x; 1.0194x over previous
import jax
import jax.numpy as jnp
from jax import lax
from jax.experimental import pallas as pl
from jax.experimental.pallas import tpu as pltpu

T = 4096
B = 16
WIN = 512
MAXN = 513
TS = 512
NT = T // TS
PADR = T + 2 * TS

_f32 = jnp.float32
_bf16 = jnp.bfloat16
_i32 = jnp.int32


def _dot(a, b):
    return lax.dot_general(a, b, (((1,), (0,)), ((), ())),
                           preferred_element_type=_f32)


def _dotT(a, b):
    return lax.dot_general(a, b, (((0,), (0,)), ((), ())),
                           preferred_element_type=_f32)


def _dotR(a, b):
    return lax.dot_general(a, b, (((0,), (1,)), ((), ())),
                           preferred_element_type=_f32)


def _prep_kernel(cu_ref,
                 kds_W1_ref, kds_b1_ref, kds_W2_ref, kds_b2_ref,
                 segc_ref, segr_ref,
                 S_ref, ohl_ref, tab_ref):
    iota_t = lax.broadcasted_iota(_i32, (1, T), 1)
    iota_n = lax.broadcasted_iota(_i32, (1, MAXN), 1)
    for s in range(B):
        c = cu_ref[s]
        n = cu_ref[s + 1]
        S_ref[s:s + 1, :] = ((iota_t >= c) & (iota_t < n)).astype(_f32)
        ohl_ref[s:s + 1, :] = (iota_n == (n - c)).astype(_f32)

    ar_col = lax.broadcasted_iota(_i32, (B, 1), 0).astype(_f32)
    segc_ref[...] = _dotT(S_ref[...], ar_col)
    segr_ref[...] = jnp.full((1, PADR), -1.0, _f32)
    segr_ref[0:1, TS:TS + T] = _dotT(ar_col, S_ref[...])

    tab_ref[...] = (_dot(jax.nn.relu(kds_W1_ref[...] + kds_b1_ref[...]),
                         kds_W2_ref[...]) + kds_b2_ref[...]).astype(_bf16)


def _rank_kernel(magc_ref, segc_ref, magr_ref, segr_ref, rank_ref):
    i = pl.program_id(0)
    mag_c = magc_ref[...]
    seg_c = segc_ref[...]
    tri = (lax.broadcasted_iota(_i32, (TS, TS), 1) <
           lax.broadcasted_iota(_i32, (TS, TS), 0))
    ones = jnp.ones((TS, 1), _bf16)
    cnt = jnp.zeros((TS, 1), _f32)
    for k in range(3):
        off = (i + k) * TS
        mag_r = magr_ref[0:1, pl.ds(off, TS)]
        seg_r = segr_ref[0:1, pl.ds(off, TS)]
        less = mag_r < mag_c
        if k == 0:
            cm = less | (mag_r == mag_c)
        elif k == 1:
            cm = less | ((mag_r == mag_c) & tri)
        else:
            cm = less
        m = (cm & (seg_r == seg_c)).astype(_bf16)
        cnt = cnt + _dot(m, ones)
    rank_ref[...] = cnt.astype(_i32)


def _ds_kernel(flat_ref, rank_ref, S_ref, tab_ref,
               vds_W1_ref, vds_b1_ref, vds_W2_ref, vds_b2_ref,
               eds_W1_ref, eds_b1_ref, eds_W2_ref, eds_b2_ref,
               km_W1b_ref, zds_ref, cseg_ref, y2ds_scr):
    i = pl.program_id(0)

    @pl.when(i == 0)
    def _():
        y2ds_scr[...] = jnp.zeros((B, 128), _f32)

    fl = flat_ref[...]
    P = (rank_ref[...] ==
         lax.broadcasted_iota(_i32, (TS, WIN), 1)).astype(_bf16)
    kds_tok = _dot(P, tab_ref[...])
    vds_tok = _dot(jax.nn.relu(_dot(fl, vds_W1_ref[...]) + vds_b1_ref[...]),
                   vds_W2_ref[...]) + vds_b2_ref[...]
    y2ds_scr[...] += _dot(S_ref[...], vds_tok * kds_tok)

    @pl.when(i == NT - 1)
    def _():
        z_ds = _dot(jax.nn.relu(_dot(y2ds_scr[...], eds_W1_ref[...]) +
                                eds_b1_ref[...]),
                    eds_W2_ref[...]) + eds_b2_ref[...]
        zds_ref[...] = z_ds
        cseg_ref[...] = _dot(z_ds, km_W1b_ref[...])


def _main_kernel(flat_ref, rank_ref, S_ref, zds_ref, cseg_ref, ohl_ref,
                 km_W1a_ref, km_b1_ref, km_W2_ref, km_b2_ref,
                 vm_W1a_ref, vm_W1b_ref, vm_b1_ref, vm_W2_ref, vm_b2_ref,
                 em_W1a_ref, em_W1b_ref, em_b1_ref, em_W2_ref, em_b2_ref,
                 out_ref, y2_scr):
    i = pl.program_id(0)

    @pl.when(i == 0)
    def _():
        y2_scr[...] = jnp.zeros((B, 64), _f32)

    fl = flat_ref[...]
    St = S_ref[...]
    P = (rank_ref[...] ==
         lax.broadcasted_iota(_i32, (TS, WIN), 1)).astype(_bf16)
    g = jax.nn.relu(_dot(P, km_W1a_ref[...]) + _dotT(St, cseg_ref[...])
                    + km_b1_ref[...])
    y_key = _dot(g, km_W2_ref[...]) + km_b2_ref[...]
    z_tok = _dotT(St, zds_ref[...])
    h = jax.nn.relu(_dot(fl, vm_W1a_ref[...]) + _dot(z_tok, vm_W1b_ref[...])
                    + vm_b1_ref[...])
    y_val = _dot(h, vm_W2_ref[...]) + vm_b2_ref[...]
    y2_scr[...] += _dot(St, y_val * y_key)

    @pl.when(i == NT - 1)
    def _():
        len_part = _dot(ohl_ref[...], em_W1b_ref[...])
        hE = jax.nn.relu(_dot(y2_scr[...], em_W1a_ref[...]) + len_part
                         + em_b1_ref[...])
        out_ref[...] = _dot(hE, em_W2_ref[...]) + em_b2_ref[...]


def _vm(block=None, imap=None):
    if block is None:
        return pl.BlockSpec(memory_space=pltpu.VMEM)
    return pl.BlockSpec(block, imap, memory_space=pltpu.VMEM)


def kernel(flat, cu_seqlens, params):
    p = params
    r2 = lambda b: b.reshape(1, -1)
    cu = cu_seqlens.astype(_i32)

    magc = flat @ p["rank_W"] + p["rank_b"]
    magr = jnp.pad(magc.reshape(1, T), ((0, 0), (TS, TS)))

    segc, segr, S, ohl, kds_tab = pl.pallas_call(
        _prep_kernel,
        out_shape=(jax.ShapeDtypeStruct((T, 1), _f32),
                   jax.ShapeDtypeStruct((1, PADR), _f32),
                   jax.ShapeDtypeStruct((B, T), _f32),
                   jax.ShapeDtypeStruct((B, MAXN), _f32),
                   jax.ShapeDtypeStruct((WIN, 128), _bf16)),
        in_specs=[pl.BlockSpec(memory_space=pltpu.SMEM)] + [_vm()] * 4,
        out_specs=(_vm(),) * 5,
    )(cu,
      p["key_ds"]["W1"][:WIN], r2(p["key_ds"]["b1"]),
      p["key_ds"]["W2"], r2(p["key_ds"]["b2"]))

    rank = pl.pallas_call(
        _rank_kernel,
        grid=(NT,),
        out_shape=jax.ShapeDtypeStruct((T, 1), _i32),
        in_specs=[_vm((TS, 1), lambda i: (i, 0)),
                  _vm((TS, 1), lambda i: (i, 0)),
                  _vm(), _vm()],
        out_specs=_vm((TS, 1), lambda i: (i, 0)),
    )(magc, segc, magr, segr)

    z_ds, c_seg = pl.pallas_call(
        _ds_kernel,
        grid=(NT,),
        out_shape=(jax.ShapeDtypeStruct((B, 128), _f32),
                   jax.ShapeDtypeStruct((B, 352), _f32)),
        in_specs=[_vm((TS, 128), lambda i: (i, 0)),
                  _vm((TS, 1), lambda i: (i, 0)),
                  _vm((B, TS), lambda i: (0, i)),
                  _vm()] + [_vm()] * 9,
        out_specs=(_vm((B, 128), lambda i: (0, 0)),
                   _vm((B, 352), lambda i: (0, 0))),
        scratch_shapes=[pltpu.VMEM((B, 128), _f32)],
    )(flat, rank, S, kds_tab,
      p["val_ds"]["W1"], r2(p["val_ds"]["b1"]),
      p["val_ds"]["W2"], r2(p["val_ds"]["b2"]),
      p["enc_ds"]["W1"], r2(p["enc_ds"]["b1"]),
      p["enc_ds"]["W2"], r2(p["enc_ds"]["b2"]),
      p["key_main"]["W1"][MAXN:])

    return pl.pallas_call(
        _main_kernel,
        grid=(NT,),
        out_shape=jax.ShapeDtypeStruct((B, 64), _f32),
        in_specs=[_vm((TS, 128), lambda i: (i, 0)),
                  _vm((TS, 1), lambda i: (i, 0)),
                  _vm((B, TS), lambda i: (0, i)),
                  _vm(), _vm(), _vm()] + [_vm()] * 14,
        out_specs=_vm((B, 64), lambda i: (0, 0)),
        scratch_shapes=[pltpu.VMEM((B, 64), _f32)],
    )(flat, rank, S, z_ds, c_seg, ohl,
      p["key_main"]["W1"][:WIN].astype(_bf16), r2(p["key_main"]["b1"]),
      p["key_main"]["W2"], r2(p["key_main"]["b2"]),
      p["val_main"]["W1"][:128], p["val_main"]["W1"][128:],
      r2(p["val_main"]["b1"]), p["val_main"]["W2"], r2(p["val_main"]["b2"]),
      p["enc_main"]["W1"][:64], p["enc_main"]["W1"][64:],
      r2(p["enc_main"]["b1"]), p["enc_main"]["W2"], r2(p["enc_main"]["b2"]))
